# Initial kernel scaffold; baseline (speedup 1.0000x reference)
import jax
import jax.numpy as jnp
from jax.experimental import pallas as pl


def kernel(node_features, pred_edge_features, assoc_edge_features, global_features, params, pred_edge_index, assoc_edge_index):
    raise NotImplementedError("write your pallas kernel here")


# speedup vs baseline: 1.7383x; 1.7383x over previous
import functools

import jax
import jax.numpy as jnp
from jax import lax
from jax.experimental import pallas as pl
from jax.experimental.pallas import tpu as pltpu
from jax.experimental.pallas import tpu_sc as plsc

N = 50000
E = 800000
D_IN = 6
D = 16

NW = 32
EPW = 25600
E_PAD = NW * EPW
NBATCH = EPW // 128
NB = 8
NCHUNK = NBATCH // NB
CHUNK = NB * 128

NROW = 50048
DUMP = 50000
RPT = NROW // 16

BE = 2048
BN = 2000


def _full_spec(shape):
    nd = len(shape)
    return pl.BlockSpec(shape, lambda i: (0,) * nd)


def _ln(y, gamma, beta):
    mu = jnp.mean(y, axis=-1, keepdims=True)
    var = jnp.mean((y - mu) ** 2, axis=-1, keepdims=True)
    return (y - mu) * lax.rsqrt(var + 1e-5) * gamma + beta


def _enc_x_body(nf, w, b, gam, bet, out):
    out[:] = _ln(jnp.dot(nf[:], w[:], preferred_element_type=jnp.float32) + b[:],
                 gam[:], bet[:])


def _enc_x(node_features, p):
    return pl.pallas_call(
        _enc_x_body,
        grid=(N // BN,),
        in_specs=[pl.BlockSpec((BN, D_IN), lambda i: (i, 0)),
                  _full_spec((D_IN, D)), _full_spec((1, D)),
                  _full_spec((1, D)), _full_spec((1, D))],
        out_specs=pl.BlockSpec((BN, D), lambda i: (i, 0)),
        out_shape=jax.ShapeDtypeStruct((N, D), jnp.float32),
    )(node_features, p["Ws"][0], p["bs"][0].reshape(1, D),
      p["gamma"].reshape(1, D), p["beta"].reshape(1, D))


def _edge1_body(xs, xd, ef, gf, we, be_, ge_, bt_, wg, bg, gg, btg,
                w0, b0, w1, b1, w2, b2, w3, b3, out):
    e1 = _ln(ef[:] * we[:] + be_[:], ge_[:], bt_[:])
    g = _ln(gf[:] * wg[:] + bg[:], gg[:], btg[:])
    h = jnp.dot(xs[:], w0[0:16], preferred_element_type=jnp.float32)
    h += jnp.dot(xd[:], w0[16:32], preferred_element_type=jnp.float32)
    h += jnp.dot(e1, w0[32:48], preferred_element_type=jnp.float32)
    h += jnp.dot(g, w0[48:64], preferred_element_type=jnp.float32) + b0[:]
    h = jax.nn.relu(h)
    h = jax.nn.relu(jnp.dot(h, w1[:], preferred_element_type=jnp.float32) + b1[:])
    h = jax.nn.relu(jnp.dot(h, w2[:], preferred_element_type=jnp.float32) + b2[:])
    out[:] = jnp.dot(h, w3[:], preferred_element_type=jnp.float32) + b3[:]


def _edge1(xs, xd, ef_pad, gfeat, pe, pg, pm):
    h = pm["Ws"][1].shape[0]
    specs = [pl.BlockSpec((BE, D), lambda i: (i, 0)),
             pl.BlockSpec((BE, D), lambda i: (i, 0)),
             pl.BlockSpec((BE, 1), lambda i: (i, 0)),
             _full_spec((1, 1)),
             _full_spec((1, D)), _full_spec((1, D)), _full_spec((1, D)), _full_spec((1, D)),
             _full_spec((1, D)), _full_spec((1, D)), _full_spec((1, D)), _full_spec((1, D)),
             _full_spec((4 * D, h)), _full_spec((1, h)),
             _full_spec((h, h)), _full_spec((1, h)),
             _full_spec((h, h)), _full_spec((1, h)),
             _full_spec((h, D)), _full_spec((1, D))]
    return pl.pallas_call(
        _edge1_body,
        grid=(E_PAD // BE,),
        in_specs=specs,
        out_specs=pl.BlockSpec((BE, D), lambda i: (i, 0)),
        out_shape=jax.ShapeDtypeStruct((E_PAD, D), jnp.float32),
    )(xs, xd, ef_pad, gfeat,
      pe["Ws"][0].reshape(1, D), pe["bs"][0].reshape(1, D),
      pe["gamma"].reshape(1, D), pe["beta"].reshape(1, D),
      pg["Ws"][0].reshape(1, D), pg["bs"][0].reshape(1, D),
      pg["gamma"].reshape(1, D), pg["beta"].reshape(1, D),
      pm["Ws"][0], pm["bs"][0].reshape(1, h),
      pm["Ws"][1], pm["bs"][1].reshape(1, h),
      pm["Ws"][2], pm["bs"][2].reshape(1, h),
      pm["Ws"][3], pm["bs"][3].reshape(1, D))


def _edge2_body(xcs, xcd, ef, we, be_, ge_, bt_,
                w0, b0, w1, b1, w2, b2, w3, b3, wd0, bd0, wd1, bd1,
                out_e, out_d):
    e2 = _ln(ef[:] * we[:] + be_[:], ge_[:], bt_[:])
    h = jnp.dot(xcs[:], w0[0:32], preferred_element_type=jnp.float32)
    h += jnp.dot(xcd[:], w0[32:64], preferred_element_type=jnp.float32)
    h += jnp.dot(e2, w0[64:80], preferred_element_type=jnp.float32) + b0[:]
    h = jax.nn.relu(h)
    h = jax.nn.relu(jnp.dot(h, w1[:], preferred_element_type=jnp.float32) + b1[:])
    h = jax.nn.relu(jnp.dot(h, w2[:], preferred_element_type=jnp.float32) + b2[:])
    eu = jnp.dot(h, w3[:], preferred_element_type=jnp.float32) + b3[:]
    out_e[:] = eu
    d = jax.nn.relu(jnp.dot(eu, wd0[:], preferred_element_type=jnp.float32) + bd0[:])
    out_d[:] = jnp.dot(d, wd1[:], preferred_element_type=jnp.float32) + bd1[:]


def _edge2(xcs, xcd, ef_pad, pe, pm, pdec):
    h = pm["Ws"][1].shape[0]
    specs = [pl.BlockSpec((BE, 2 * D), lambda i: (i, 0)),
             pl.BlockSpec((BE, 2 * D), lambda i: (i, 0)),
             pl.BlockSpec((BE, 1), lambda i: (i, 0)),
             _full_spec((1, D)), _full_spec((1, D)), _full_spec((1, D)), _full_spec((1, D)),
             _full_spec((5 * D, h)), _full_spec((1, h)),
             _full_spec((h, h)), _full_spec((1, h)),
             _full_spec((h, h)), _full_spec((1, h)),
             _full_spec((h, D)), _full_spec((1, D)),
             _full_spec((D, D)), _full_spec((1, D)),
             _full_spec((D, 1)), _full_spec((1, 1))]
    return pl.pallas_call(
        _edge2_body,
        grid=(E_PAD // BE,),
        in_specs=specs,
        out_specs=[pl.BlockSpec((BE, D), lambda i: (i, 0)),
                   pl.BlockSpec((BE, 1), lambda i: (i, 0))],
        out_shape=[jax.ShapeDtypeStruct((E_PAD, D), jnp.float32),
                   jax.ShapeDtypeStruct((E_PAD, 1), jnp.float32)],
    )(xcs, xcd, ef_pad,
      pe["Ws"][0].reshape(1, D), pe["bs"][0].reshape(1, D),
      pe["gamma"].reshape(1, D), pe["beta"].reshape(1, D),
      pm["Ws"][0], pm["bs"][0].reshape(1, h),
      pm["Ws"][1], pm["bs"][1].reshape(1, h),
      pm["Ws"][2], pm["bs"][2].reshape(1, h),
      pm["Ws"][3], pm["bs"][3].reshape(1, D),
      pdec["Ws"][0], pdec["bs"][0].reshape(1, D),
      pdec["Ws"][1], pdec["bs"][1].reshape(1, 1))


def _node1_body(x, p0, p1, gf, wg, bg, gg, btg, w0, b0, w1, b1, w2, b2, out):
    agg = p0[:] + p1[:]
    g = _ln(gf[:] * wg[:] + bg[:], gg[:], btg[:])
    h = jnp.dot(x[:], w0[0:16], preferred_element_type=jnp.float32)
    h += jnp.dot(agg, w0[16:32], preferred_element_type=jnp.float32)
    h += jnp.dot(g, w0[32:48], preferred_element_type=jnp.float32) + b0[:]
    h = jax.nn.relu(h)
    h = jax.nn.relu(jnp.dot(h, w1[:], preferred_element_type=jnp.float32) + b1[:])
    x1 = jnp.dot(h, w2[:], preferred_element_type=jnp.float32) + b2[:]
    out[:] = jnp.concatenate([x1, x[:]], axis=-1)


def _node1(x, parts, gfeat, pg, pm):
    h = pm["Ws"][1].shape[0]
    specs = [pl.BlockSpec((BN, D), lambda i: (i, 0)),
             pl.BlockSpec((BN, D), lambda i: (i, 0)),
             pl.BlockSpec((BN, D), lambda i: (i, 0)),
             _full_spec((1, 1)),
             _full_spec((1, D)), _full_spec((1, D)), _full_spec((1, D)), _full_spec((1, D)),
             _full_spec((3 * D, h)), _full_spec((1, h)),
             _full_spec((h, h)), _full_spec((1, h)),
             _full_spec((h, D)), _full_spec((1, D))]
    return pl.pallas_call(
        _node1_body,
        grid=(N // BN,),
        in_specs=specs,
        out_specs=pl.BlockSpec((BN, 2 * D), lambda i: (i, 0)),
        out_shape=jax.ShapeDtypeStruct((N, 2 * D), jnp.float32),
    )(x, parts[0], parts[1], gfeat,
      pg["Ws"][0].reshape(1, D), pg["bs"][0].reshape(1, D),
      pg["gamma"].reshape(1, D), pg["beta"].reshape(1, D),
      pm["Ws"][0], pm["bs"][0].reshape(1, h),
      pm["Ws"][1], pm["bs"][1].reshape(1, h),
      pm["Ws"][2], pm["bs"][2].reshape(1, D))


def _node2_body(xc, p0, p1, w0, b0, w1, b1, w2, b2, wn0, bn0, wn1, bn1, out):
    agg = p0[:] + p1[:]
    h = jnp.dot(xc[:], w0[0:32], preferred_element_type=jnp.float32)
    h += jnp.dot(agg, w0[32:48], preferred_element_type=jnp.float32) + b0[:]
    h = jax.nn.relu(h)
    h = jax.nn.relu(jnp.dot(h, w1[:], preferred_element_type=jnp.float32) + b1[:])
    x2 = jnp.dot(h, w2[:], preferred_element_type=jnp.float32) + b2[:]
    d = jax.nn.relu(jnp.dot(x2, wn0[:], preferred_element_type=jnp.float32) + bn0[:])
    out[:] = jnp.dot(d, wn1[:], preferred_element_type=jnp.float32) + bn1[:]


def _node2(xc, parts, pm, pdec):
    h = pm["Ws"][1].shape[0]
    specs = [pl.BlockSpec((BN, 2 * D), lambda i: (i, 0)),
             pl.BlockSpec((BN, D), lambda i: (i, 0)),
             pl.BlockSpec((BN, D), lambda i: (i, 0)),
             _full_spec((3 * D, h)), _full_spec((1, h)),
             _full_spec((h, h)), _full_spec((1, h)),
             _full_spec((h, D)), _full_spec((1, D)),
             _full_spec((D, D)), _full_spec((1, D)),
             _full_spec((D, D_IN)), _full_spec((1, D_IN))]
    return pl.pallas_call(
        _node2_body,
        grid=(N // BN,),
        in_specs=specs,
        out_specs=pl.BlockSpec((BN, D_IN), lambda i: (i, 0)),
        out_shape=jax.ShapeDtypeStruct((N, D_IN), jnp.float32),
    )(xc, parts[0], parts[1],
      pm["Ws"][0], pm["bs"][0].reshape(1, h),
      pm["Ws"][1], pm["bs"][1].reshape(1, h),
      pm["Ws"][2], pm["bs"][2].reshape(1, D),
      pdec["Ws"][0], pdec["bs"][0].reshape(1, D),
      pdec["Ws"][1], pdec["bs"][1].reshape(1, D_IN))


@functools.lru_cache(maxsize=None)
def _make_gather(d):
    mesh = plsc.VectorSubcoreMesh(core_axis_name="c", subcore_axis_name="s",
                                  num_cores=2, num_subcores=16)

    @functools.partial(
        pl.kernel,
        out_type=[jax.ShapeDtypeStruct((E_PAD, d), jnp.float32),
                  jax.ShapeDtypeStruct((E_PAD, d), jnp.float32)],
        mesh=mesh,
        compiler_params=pltpu.CompilerParams(use_tc_tiling_on_sc=False),
        scratch_types=[pltpu.VMEM((NB, 128), jnp.int32),
                       pltpu.VMEM((CHUNK, d), jnp.float32),
                       pltpu.SemaphoreType.DMA],
    )
    def gath(tab, src_i, dst_i, out_s, out_d, idx_v, rows_v, sem):
        wid = lax.axis_index("c") * 16 + lax.axis_index("s")
        bb = wid * NBATCH
        be = wid * EPW

        def chunk(jj, idx_hbm, out_hbm):
            pltpu.sync_copy(idx_hbm.at[pl.ds(bb + jj * NB, NB)], idx_v)
            descs = [pltpu.async_copy(tab.at[idx_v.at[b]],
                                      rows_v.at[pl.ds(b * 128, 128)], sem)
                     for b in range(NB)]
            for dd in descs:
                dd.wait()
            pltpu.sync_copy(rows_v, out_hbm.at[pl.ds(be + jj * CHUNK, CHUNK)])

        def body(jj, _):
            chunk(jj, src_i, out_s)
            chunk(jj, dst_i, out_d)
            return 0

        lax.fori_loop(0, NCHUNK, body, 0)

    return gath


@functools.lru_cache(maxsize=None)
def _make_scatter():
    mesh = plsc.VectorSubcoreMesh(core_axis_name="c", subcore_axis_name="s",
                                  num_cores=2, num_subcores=16)

    @functools.partial(
        pl.kernel,
        out_type=jax.ShapeDtypeStruct((2, NROW, D), jnp.float32),
        mesh=mesh,
        compiler_params=pltpu.CompilerParams(use_tc_tiling_on_sc=False),
        scratch_types=[pltpu.VMEM((NB, 128), jnp.int32),
                       pltpu.VMEM((CHUNK, D), jnp.float32),
                       pltpu.VMEM((RPT, D), jnp.float32),
                       pltpu.VMEM_SHARED((NROW, D), jnp.float32)],
    )
    def scat(eu, dst_i, zrow, out, idx_v, rows_v, zbuf, acc):
        c = lax.axis_index("c")
        s = lax.axis_index("s")
        wid = c * 16 + s
        bb = wid * NBATCH
        be = wid * EPW

        pltpu.sync_copy(zrow.at[pl.ds(0, RPT)], zbuf)
        pltpu.sync_copy(zbuf, acc.at[pl.ds(s * RPT, RPT)])
        plsc.subcore_barrier()

        def body(jj, _):
            pltpu.sync_copy(dst_i.at[pl.ds(bb + jj * NB, NB)], idx_v)
            pltpu.sync_copy(eu.at[pl.ds(be + jj * CHUNK, CHUNK)], rows_v)
            for b in range(NB):
                pltpu.sync_copy(rows_v.at[pl.ds(b * 128, 128)],
                                acc.at[idx_v.at[b]], add=True)
            return 0

        lax.fori_loop(0, NCHUNK, body, 0)
        plsc.subcore_barrier()

        pltpu.sync_copy(acc.at[pl.ds(s * RPT, RPT)], zbuf)
        pltpu.sync_copy(zbuf, out.at[c, pl.ds(s * RPT, RPT)])

    return scat


def kernel(node_features, pred_edge_features, assoc_edge_features,
           global_features, params, pred_edge_index, assoc_edge_index):
    f32 = jnp.float32
    pad = E_PAD - E

    src1 = jnp.concatenate([pred_edge_index[0], jnp.zeros((pad,), jnp.int32)])
    dst1 = jnp.concatenate([pred_edge_index[1],
                            jnp.full((pad,), DUMP, jnp.int32)])
    src2 = jnp.concatenate([assoc_edge_index[0], jnp.zeros((pad,), jnp.int32)])
    dst2 = jnp.concatenate([assoc_edge_index[1],
                            jnp.full((pad,), DUMP, jnp.int32)])
    src1 = src1.reshape(E_PAD // 128, 128)
    dst1 = dst1.reshape(E_PAD // 128, 128)
    src2 = src2.reshape(E_PAD // 128, 128)
    dst2 = dst2.reshape(E_PAD // 128, 128)

    pef = jnp.concatenate([pred_edge_features, jnp.zeros((pad, 1), f32)])
    aef = jnp.concatenate([assoc_edge_features, jnp.zeros((pad, 1), f32)])
    zrow = jnp.zeros((RPT, D), f32)

    x = _enc_x(node_features, params["node_enc"])

    xs1, xd1 = _make_gather(D)(x, src1, dst1)
    e1u = _edge1(xs1, xd1, pef, global_features,
                 params["pe_enc"], params["g_enc"], params["tgl1_edge"])
    parts1 = _make_scatter()(e1u, dst1, zrow)[:, :N, :]
    xc = _node1(x, parts1, global_features, params["g_enc"],
                params["tgl1_node"])

    xcs, xcd = _make_gather(2 * D)(xc, src2, dst2)
    e2u, eo = _edge2(xcs, xcd, aef, params["ae_enc"], params["tgl2_edge"],
                     params["edge_dec"])
    parts2 = _make_scatter()(e2u, dst2, zrow)[:, :N, :]
    nodes_out = _node2(xc, parts2, params["tgl2_node"], params["node_dec"])

    return nodes_out, eo[:E]


# speedup vs baseline: 1.8392x; 1.0581x over previous
import functools

import jax
import jax.numpy as jnp
from jax import lax
from jax.experimental import pallas as pl
from jax.experimental.pallas import tpu as pltpu
from jax.experimental.pallas import tpu_sc as plsc

N = 50000
E = 800000
D_IN = 6
D = 16

NW = 32
NSLICE = 5
E_PAD = 819200
E_S = E_PAD // NSLICE
EPW = E_S // NW
NBATCH = EPW // 128
NB = 8
NCHUNK = NBATCH // NB
CHUNK = NB * 128
IDXR = E_S // 128

NROW = 50048
DUMP = 50000
RPT = NROW // 16

BE = 2048
BN = 2000


def _full_spec(shape):
    nd = len(shape)
    return pl.BlockSpec(shape, lambda i: (0,) * nd)


def _dot(a, w):
    return jnp.dot(a, w, preferred_element_type=jnp.float32)


def _ln(y, gamma, beta):
    mu = jnp.mean(y, axis=-1, keepdims=True)
    var = jnp.mean((y - mu) ** 2, axis=-1, keepdims=True)
    return (y - mu) * lax.rsqrt(var + 1e-5) * gamma + beta


def _enc_x_body(nf, w, b, gam, bet, out):
    out[:] = _ln(_dot(nf[:], w[:]) + b[:], gam[:], bet[:])


def _enc_x(node_features, p):
    return pl.pallas_call(
        _enc_x_body,
        grid=(N // BN,),
        in_specs=[pl.BlockSpec((BN, D_IN), lambda i: (i, 0)),
                  _full_spec((D_IN, D)), _full_spec((1, D)),
                  _full_spec((1, D)), _full_spec((1, D))],
        out_specs=pl.BlockSpec((BN, D), lambda i: (i, 0)),
        out_shape=jax.ShapeDtypeStruct((N, D), jnp.float32),
    )(node_features, p["Ws"][0], p["bs"][0].reshape(1, D),
      p["gamma"].reshape(1, D), p["beta"].reshape(1, D))


def _edge1_body(xs, xd, ef, gf, we, be_, ge_, bt_, wg, bg, gg, btg,
                w0, b0, w1, b1, w2, b2, w3, b3, out):
    e1 = _ln(ef[:] * we[:] + be_[:], ge_[:], bt_[:])
    g = _ln(gf[:] * wg[:] + bg[:], gg[:], btg[:])
    h = _dot(xs[:], w0[0:16])
    h += _dot(xd[:], w0[16:32])
    h += _dot(e1, w0[32:48])
    h += _dot(g, w0[48:64]) + b0[:]
    h = jax.nn.relu(h)
    h = jax.nn.relu(_dot(h, w1[:]) + b1[:])
    h = jax.nn.relu(_dot(h, w2[:]) + b2[:])
    out[:] = _dot(h, w3[:]) + b3[:]


def _edge1(xs, xd, ef, gfeat, pe, pg, pm):
    h = pm["Ws"][1].shape[0]
    specs = [pl.BlockSpec((BE, D), lambda i: (i, 0)),
             pl.BlockSpec((BE, D), lambda i: (i, 0)),
             pl.BlockSpec((BE, 1), lambda i: (i, 0)),
             _full_spec((1, 1)),
             _full_spec((1, D)), _full_spec((1, D)), _full_spec((1, D)), _full_spec((1, D)),
             _full_spec((1, D)), _full_spec((1, D)), _full_spec((1, D)), _full_spec((1, D)),
             _full_spec((4 * D, h)), _full_spec((1, h)),
             _full_spec((h, h)), _full_spec((1, h)),
             _full_spec((h, h)), _full_spec((1, h)),
             _full_spec((h, D)), _full_spec((1, D))]
    return pl.pallas_call(
        _edge1_body,
        grid=(E_S // BE,),
        in_specs=specs,
        out_specs=pl.BlockSpec((BE, D), lambda i: (i, 0)),
        out_shape=jax.ShapeDtypeStruct((E_S, D), jnp.float32),
    )(xs, xd, ef, gfeat,
      pe["Ws"][0].reshape(1, D), pe["bs"][0].reshape(1, D),
      pe["gamma"].reshape(1, D), pe["beta"].reshape(1, D),
      pg["Ws"][0].reshape(1, D), pg["bs"][0].reshape(1, D),
      pg["gamma"].reshape(1, D), pg["beta"].reshape(1, D),
      pm["Ws"][0], pm["bs"][0].reshape(1, h),
      pm["Ws"][1], pm["bs"][1].reshape(1, h),
      pm["Ws"][2], pm["bs"][2].reshape(1, h),
      pm["Ws"][3], pm["bs"][3].reshape(1, D))


def _edge2_body(xcs, xcd, ef, we, be_, ge_, bt_,
                w0, b0, w1, b1, w2, b2, w3, b3, wd0, bd0, wd1, bd1,
                out_e, out_d):
    e2 = _ln(ef[:] * we[:] + be_[:], ge_[:], bt_[:])
    h = _dot(xcs[:], w0[0:32])
    h += _dot(xcd[:], w0[32:64])
    h += _dot(e2, w0[64:80]) + b0[:]
    h = jax.nn.relu(h)
    h = jax.nn.relu(_dot(h, w1[:]) + b1[:])
    h = jax.nn.relu(_dot(h, w2[:]) + b2[:])
    eu = _dot(h, w3[:]) + b3[:]
    out_e[:] = eu
    d = jax.nn.relu(_dot(eu, wd0[:]) + bd0[:])
    out_d[:] = _dot(d, wd1[:]) + bd1[:]


def _edge2(xcs, xcd, ef, pe, pm, pdec):
    h = pm["Ws"][1].shape[0]
    specs = [pl.BlockSpec((BE, 2 * D), lambda i: (i, 0)),
             pl.BlockSpec((BE, 2 * D), lambda i: (i, 0)),
             pl.BlockSpec((BE, 1), lambda i: (i, 0)),
             _full_spec((1, D)), _full_spec((1, D)), _full_spec((1, D)), _full_spec((1, D)),
             _full_spec((5 * D, h)), _full_spec((1, h)),
             _full_spec((h, h)), _full_spec((1, h)),
             _full_spec((h, h)), _full_spec((1, h)),
             _full_spec((h, D)), _full_spec((1, D)),
             _full_spec((D, D)), _full_spec((1, D)),
             _full_spec((D, 1)), _full_spec((1, 1))]
    return pl.pallas_call(
        _edge2_body,
        grid=(E_S // BE,),
        in_specs=specs,
        out_specs=[pl.BlockSpec((BE, D), lambda i: (i, 0)),
                   pl.BlockSpec((BE, 1), lambda i: (i, 0))],
        out_shape=[jax.ShapeDtypeStruct((E_S, D), jnp.float32),
                   jax.ShapeDtypeStruct((E_S, 1), jnp.float32)],
    )(xcs, xcd, ef,
      pe["Ws"][0].reshape(1, D), pe["bs"][0].reshape(1, D),
      pe["gamma"].reshape(1, D), pe["beta"].reshape(1, D),
      pm["Ws"][0], pm["bs"][0].reshape(1, h),
      pm["Ws"][1], pm["bs"][1].reshape(1, h),
      pm["Ws"][2], pm["bs"][2].reshape(1, h),
      pm["Ws"][3], pm["bs"][3].reshape(1, D),
      pdec["Ws"][0], pdec["bs"][0].reshape(1, D),
      pdec["Ws"][1], pdec["bs"][1].reshape(1, 1))


def _node1_body(x, p0, p1, gf, wg, bg, gg, btg, w0, b0, w1, b1, w2, b2, out):
    agg = p0[:] + p1[:]
    g = _ln(gf[:] * wg[:] + bg[:], gg[:], btg[:])
    h = _dot(x[:], w0[0:16])
    h += _dot(agg, w0[16:32])
    h += _dot(g, w0[32:48]) + b0[:]
    h = jax.nn.relu(h)
    h = jax.nn.relu(_dot(h, w1[:]) + b1[:])
    x1 = _dot(h, w2[:]) + b2[:]
    out[:] = jnp.concatenate([x1, x[:]], axis=-1)


def _node1(x, parts, gfeat, pg, pm):
    h = pm["Ws"][1].shape[0]
    specs = [pl.BlockSpec((BN, D), lambda i: (i, 0)),
             pl.BlockSpec((BN, D), lambda i: (i, 0)),
             pl.BlockSpec((BN, D), lambda i: (i, 0)),
             _full_spec((1, 1)),
             _full_spec((1, D)), _full_spec((1, D)), _full_spec((1, D)), _full_spec((1, D)),
             _full_spec((3 * D, h)), _full_spec((1, h)),
             _full_spec((h, h)), _full_spec((1, h)),
             _full_spec((h, D)), _full_spec((1, D))]
    return pl.pallas_call(
        _node1_body,
        grid=(N // BN,),
        in_specs=specs,
        out_specs=pl.BlockSpec((BN, 2 * D), lambda i: (i, 0)),
        out_shape=jax.ShapeDtypeStruct((N, 2 * D), jnp.float32),
    )(x, parts[0], parts[1], gfeat,
      pg["Ws"][0].reshape(1, D), pg["bs"][0].reshape(1, D),
      pg["gamma"].reshape(1, D), pg["beta"].reshape(1, D),
      pm["Ws"][0], pm["bs"][0].reshape(1, h),
      pm["Ws"][1], pm["bs"][1].reshape(1, h),
      pm["Ws"][2], pm["bs"][2].reshape(1, D))


def _node2_body(xc, p0, p1, w0, b0, w1, b1, w2, b2, wn0, bn0, wn1, bn1, out):
    agg = p0[:] + p1[:]
    h = _dot(xc[:], w0[0:32])
    h += _dot(agg, w0[32:48]) + b0[:]
    h = jax.nn.relu(h)
    h = jax.nn.relu(_dot(h, w1[:]) + b1[:])
    x2 = _dot(h, w2[:]) + b2[:]
    d = jax.nn.relu(_dot(x2, wn0[:]) + bn0[:])
    out[:] = _dot(d, wn1[:]) + bn1[:]


def _node2(xc, parts, pm, pdec):
    h = pm["Ws"][1].shape[0]
    specs = [pl.BlockSpec((BN, 2 * D), lambda i: (i, 0)),
             pl.BlockSpec((BN, D), lambda i: (i, 0)),
             pl.BlockSpec((BN, D), lambda i: (i, 0)),
             _full_spec((3 * D, h)), _full_spec((1, h)),
             _full_spec((h, h)), _full_spec((1, h)),
             _full_spec((h, D)), _full_spec((1, D)),
             _full_spec((D, D)), _full_spec((1, D)),
             _full_spec((D, D_IN)), _full_spec((1, D_IN))]
    return pl.pallas_call(
        _node2_body,
        grid=(N // BN,),
        in_specs=specs,
        out_specs=pl.BlockSpec((BN, D_IN), lambda i: (i, 0)),
        out_shape=jax.ShapeDtypeStruct((N, D_IN), jnp.float32),
    )(xc, parts[0], parts[1],
      pm["Ws"][0], pm["bs"][0].reshape(1, h),
      pm["Ws"][1], pm["bs"][1].reshape(1, h),
      pm["Ws"][2], pm["bs"][2].reshape(1, D),
      pdec["Ws"][0], pdec["bs"][0].reshape(1, D),
      pdec["Ws"][1], pdec["bs"][1].reshape(1, D_IN))


@functools.lru_cache(maxsize=None)
def _make_gather(d):
    mesh = plsc.VectorSubcoreMesh(core_axis_name="c", subcore_axis_name="s",
                                  num_cores=2, num_subcores=16)

    @functools.partial(
        pl.kernel,
        out_type=[jax.ShapeDtypeStruct((E_S, d), jnp.float32),
                  jax.ShapeDtypeStruct((E_S, d), jnp.float32)],
        mesh=mesh,
        compiler_params=pltpu.CompilerParams(use_tc_tiling_on_sc=False),
        scratch_types=[pltpu.VMEM((NB, 128), jnp.int32),
                       pltpu.VMEM((CHUNK, d), jnp.float32),
                       pltpu.SemaphoreType.DMA],
    )
    def gath(tab, src_i, dst_i, out_s, out_d, idx_v, rows_v, sem):
        wid = lax.axis_index("c") * 16 + lax.axis_index("s")
        bb = wid * NBATCH
        be = wid * EPW

        def chunk(jj, idx_hbm, out_hbm):
            pltpu.sync_copy(idx_hbm.at[pl.ds(bb + jj * NB, NB)], idx_v)
            descs = [pltpu.async_copy(tab.at[idx_v.at[b]],
                                      rows_v.at[pl.ds(b * 128, 128)], sem)
                     for b in range(NB)]
            for dd in descs:
                dd.wait()
            pltpu.sync_copy(rows_v, out_hbm.at[pl.ds(be + jj * CHUNK, CHUNK)])

        def body(jj, _):
            chunk(jj, src_i, out_s)
            chunk(jj, dst_i, out_d)
            return 0

        lax.fori_loop(0, NCHUNK, body, 0)

    return gath


@functools.lru_cache(maxsize=None)
def _make_scatter():
    mesh = plsc.VectorSubcoreMesh(core_axis_name="c", subcore_axis_name="s",
                                  num_cores=2, num_subcores=16)

    @functools.partial(
        pl.kernel,
        out_type=jax.ShapeDtypeStruct((2, NROW, D), jnp.float32),
        mesh=mesh,
        compiler_params=pltpu.CompilerParams(use_tc_tiling_on_sc=False),
        scratch_types=[pltpu.VMEM((NB, 128), jnp.int32),
                       pltpu.VMEM((CHUNK, D), jnp.float32),
                       pltpu.VMEM((RPT, D), jnp.float32),
                       pltpu.VMEM_SHARED((NROW, D), jnp.float32)],
    )
    def scat(eu0, eu1, eu2, eu3, eu4, dst_i, zrow, out,
             idx_v, rows_v, zbuf, acc):
        c = lax.axis_index("c")
        s = lax.axis_index("s")
        wid = c * 16 + s

        pltpu.sync_copy(zrow.at[pl.ds(0, RPT)], zbuf)
        pltpu.sync_copy(zbuf, acc.at[pl.ds(s * RPT, RPT)])
        plsc.subcore_barrier()

        for si, eu in enumerate((eu0, eu1, eu2, eu3, eu4)):
            bb = si * IDXR + wid * NBATCH
            be = wid * EPW

            def body(jj, _, eu=eu, bb=bb, be=be):
                pltpu.sync_copy(dst_i.at[pl.ds(bb + jj * NB, NB)], idx_v)
                pltpu.sync_copy(eu.at[pl.ds(be + jj * CHUNK, CHUNK)], rows_v)
                for b in range(NB):
                    pltpu.sync_copy(rows_v.at[pl.ds(b * 128, 128)],
                                    acc.at[idx_v.at[b]], add=True)
                return 0

            lax.fori_loop(0, NCHUNK, body, 0)
        plsc.subcore_barrier()

        pltpu.sync_copy(acc.at[pl.ds(s * RPT, RPT)], zbuf)
        pltpu.sync_copy(zbuf, out.at[c, pl.ds(s * RPT, RPT)])

    return scat


def kernel(node_features, pred_edge_features, assoc_edge_features,
           global_features, params, pred_edge_index, assoc_edge_index):
    f32 = jnp.float32
    pad = E_PAD - E

    src1 = jnp.concatenate([pred_edge_index[0], jnp.zeros((pad,), jnp.int32)])
    dst1 = jnp.concatenate([pred_edge_index[1],
                            jnp.full((pad,), DUMP, jnp.int32)])
    src2 = jnp.concatenate([assoc_edge_index[0], jnp.zeros((pad,), jnp.int32)])
    dst2 = jnp.concatenate([assoc_edge_index[1],
                            jnp.full((pad,), DUMP, jnp.int32)])
    src1 = src1.reshape(E_PAD // 128, 128)
    dst1 = dst1.reshape(E_PAD // 128, 128)
    src2 = src2.reshape(E_PAD // 128, 128)
    dst2 = dst2.reshape(E_PAD // 128, 128)

    pef = jnp.concatenate([pred_edge_features, jnp.zeros((pad, 1), f32)])
    aef = jnp.concatenate([assoc_edge_features, jnp.zeros((pad, 1), f32)])
    zrow = jnp.zeros((RPT, D), f32)

    x = _enc_x(node_features, params["node_enc"])

    g16 = _make_gather(D)
    g32 = _make_gather(2 * D)
    scat = _make_scatter()

    e1u = []
    for si in range(NSLICE):
        r0 = si * IDXR
        xs, xd = g16(x, src1[r0:r0 + IDXR], dst1[r0:r0 + IDXR])
        e1u.append(_edge1(xs, xd, pef[si * E_S:(si + 1) * E_S],
                          global_features, params["pe_enc"],
                          params["g_enc"], params["tgl1_edge"]))
    parts1 = scat(*e1u, dst1, zrow)[:, :N, :]
    xc = _node1(x, parts1, global_features, params["g_enc"],
                params["tgl1_node"])

    e2u, eo = [], []
    for si in range(NSLICE):
        r0 = si * IDXR
        xcs, xcd = g32(xc, src2[r0:r0 + IDXR], dst2[r0:r0 + IDXR])
        a, b = _edge2(xcs, xcd, aef[si * E_S:(si + 1) * E_S],
                      params["ae_enc"], params["tgl2_edge"],
                      params["edge_dec"])
        e2u.append(a)
        eo.append(b)
    parts2 = scat(*e2u, dst2, zrow)[:, :N, :]
    nodes_out = _node2(xc, parts2, params["tgl2_node"], params["node_dec"])

    return nodes_out, jnp.concatenate(eo)[:E]


# speedup vs baseline: 3.5894x; 1.9516x over previous
import functools

import jax
import jax.numpy as jnp
from jax import lax
from jax.experimental import pallas as pl
from jax.experimental.pallas import tpu as pltpu
from jax.experimental.pallas import tpu_sc as plsc

N = 50000
E = 800000
D_IN = 6
D = 16

NW = 32
NSLICE = 5
E_PAD = 819200
E_S = E_PAD // NSLICE
EPW = E_S // NW
NBATCH = EPW // 128
NB = 8
NCHUNK = NBATCH // NB
CHUNK = NB * 128
IDXR = E_S // 128

NROW = 50048
DUMP = 50000
RPT = NROW // 16

BE = 8192
BR = BE // 8
BR2 = BE // 4
BN = 2000


def _full_spec(shape):
    nd = len(shape)
    return pl.BlockSpec(shape, lambda i: (0,) * nd)


def _dot(a, w):
    return jnp.dot(a, w, preferred_element_type=jnp.float32)


def _ln(y, gamma, beta):
    mu = jnp.mean(y, axis=-1, keepdims=True)
    var = jnp.mean((y - mu) ** 2, axis=-1, keepdims=True)
    return (y - mu) * lax.rsqrt(var + 1e-5) * gamma + beta


def _enc_x_body(nf, w, b, gam, bet, out):
    out[:] = _ln(_dot(nf[:], w[:]) + b[:], gam[:], bet[:])


def _enc_x(node_features, p):
    return pl.pallas_call(
        _enc_x_body,
        grid=(N // BN,),
        in_specs=[pl.BlockSpec((BN, D_IN), lambda i: (i, 0)),
                  _full_spec((D_IN, D)), _full_spec((1, D)),
                  _full_spec((1, D)), _full_spec((1, D))],
        out_specs=pl.BlockSpec((BN, D), lambda i: (i, 0)),
        out_shape=jax.ShapeDtypeStruct((N, D), jnp.float32),
    )(node_features, p["Ws"][0], p["bs"][0].reshape(1, D),
      p["gamma"].reshape(1, D), p["beta"].reshape(1, D))


def _ln_t(y, gamma, beta):
    mu = jnp.mean(y, axis=0, keepdims=True)
    var = jnp.mean((y - mu) ** 2, axis=0, keepdims=True)
    return (y - mu) * lax.rsqrt(var + 1e-5) * gamma + beta


def _edge1_body(xs, xd, ef, gf, we, be_, ge_, bt_, wg, bg, gg, btg,
                w0x, w0d, w0e, w0g, b0, w1, b1, w2, b2, w3, b3, out):
    xst = jnp.swapaxes(xs[:], 0, 1)
    xdt = jnp.swapaxes(xd[:], 0, 1)
    pt = jnp.swapaxes(ef[:], 0, 1)
    g = _ln_t(gf[:] * wg[:] + bg[:], gg[:], btg[:])
    gc = _dot(w0g[:], g)
    subs = []
    for a in range(8):
        e1 = _ln_t(pt[a:a + 1, :] * we[:] + be_[:], ge_[:], bt_[:])
        h = _dot(w0x[:], xst[16 * a:16 * a + 16, :])
        h += _dot(w0d[:], xdt[16 * a:16 * a + 16, :])
        h += _dot(w0e[:], e1)
        h += gc + b0[:]
        h = jax.nn.relu(h)
        h = jax.nn.relu(_dot(w1[:], h) + b1[:])
        h = jax.nn.relu(_dot(w2[:], h) + b2[:])
        subs.append(_dot(w3[:], h) + b3[:])
    out[:] = jnp.swapaxes(jnp.concatenate(subs, axis=0), 0, 1)


def _edge1(xs_pk, xd_pk, ef_pk, gfeat, pe, pg, pm):
    h = pm["Ws"][1].shape[0]
    w0 = pm["Ws"][0]
    specs = [pl.BlockSpec((BR, 128), lambda i: (i, 0)),
             pl.BlockSpec((BR, 128), lambda i: (i, 0)),
             pl.BlockSpec((BR, 8), lambda i: (i, 0)),
             _full_spec((1, 1)),
             _full_spec((D, 1)), _full_spec((D, 1)), _full_spec((D, 1)), _full_spec((D, 1)),
             _full_spec((D, 1)), _full_spec((D, 1)), _full_spec((D, 1)), _full_spec((D, 1)),
             _full_spec((h, D)), _full_spec((h, D)), _full_spec((h, D)), _full_spec((h, D)),
             _full_spec((h, 1)),
             _full_spec((h, h)), _full_spec((h, 1)),
             _full_spec((h, h)), _full_spec((h, 1)),
             _full_spec((D, h)), _full_spec((D, 1))]
    return pl.pallas_call(
        _edge1_body,
        grid=(E_S // BE,),
        in_specs=specs,
        out_specs=pl.BlockSpec((BR, 128), lambda i: (i, 0)),
        out_shape=jax.ShapeDtypeStruct((E_S // 8, 128), jnp.float32),
    )(xs_pk, xd_pk, ef_pk, gfeat,
      pe["Ws"][0].reshape(D, 1), pe["bs"][0].reshape(D, 1),
      pe["gamma"].reshape(D, 1), pe["beta"].reshape(D, 1),
      pg["Ws"][0].reshape(D, 1), pg["bs"][0].reshape(D, 1),
      pg["gamma"].reshape(D, 1), pg["beta"].reshape(D, 1),
      w0[0:16].T, w0[16:32].T, w0[32:48].T, w0[48:64].T,
      pm["bs"][0].reshape(h, 1),
      pm["Ws"][1].T, pm["bs"][1].reshape(h, 1),
      pm["Ws"][2].T, pm["bs"][2].reshape(h, 1),
      pm["Ws"][3].T, pm["bs"][3].reshape(D, 1))


def _edge2_body(xcs, xcd, ef, we, be_, ge_, bt_,
                w0s, w0d, w0e, b0, w1, b1, w2, b2, w3, b3, wd0, bd0, wd1, bd1,
                out_e, out_d):
    xst = jnp.swapaxes(xcs[:], 0, 1)
    xdt = jnp.swapaxes(xcd[:], 0, 1)
    pt = jnp.swapaxes(ef[:], 0, 1)
    eus, eos = [], []
    for a in range(4):
        e2 = _ln_t(pt[a:a + 1, :] * we[:] + be_[:], ge_[:], bt_[:])
        h = _dot(w0s[:], xst[32 * a:32 * a + 32, :])
        h += _dot(w0d[:], xdt[32 * a:32 * a + 32, :])
        h += _dot(w0e[:], e2) + b0[:]
        h = jax.nn.relu(h)
        h = jax.nn.relu(_dot(w1[:], h) + b1[:])
        h = jax.nn.relu(_dot(w2[:], h) + b2[:])
        eu = _dot(w3[:], h) + b3[:]
        d = jax.nn.relu(_dot(wd0[:], eu) + bd0[:])
        eo = _dot(wd1[:], d) + bd1[:]
        eus.append(eu)
        eos.append(eo)
    half = BR2 // 2
    epk = [eus[a][:, hh * half:(hh + 1) * half]
           for a in range(4) for hh in (0, 1)]
    opk = [eos[a][:, hh * half:(hh + 1) * half]
           for a in range(4) for hh in (0, 1)]
    out_e[:] = jnp.swapaxes(jnp.concatenate(epk, axis=0), 0, 1)
    out_d[:] = jnp.swapaxes(jnp.concatenate(opk, axis=0), 0, 1)


def _edge2(xcs_pk, xcd_pk, ef_pk, pe, pm, pdec):
    h = pm["Ws"][1].shape[0]
    w0 = pm["Ws"][0]
    specs = [pl.BlockSpec((BR2, 128), lambda i: (i, 0)),
             pl.BlockSpec((BR2, 128), lambda i: (i, 0)),
             pl.BlockSpec((BR2, 4), lambda i: (i, 0)),
             _full_spec((D, 1)), _full_spec((D, 1)), _full_spec((D, 1)), _full_spec((D, 1)),
             _full_spec((h, 2 * D)), _full_spec((h, 2 * D)), _full_spec((h, D)),
             _full_spec((h, 1)),
             _full_spec((h, h)), _full_spec((h, 1)),
             _full_spec((h, h)), _full_spec((h, 1)),
             _full_spec((D, h)), _full_spec((D, 1)),
             _full_spec((D, D)), _full_spec((D, 1)),
             _full_spec((1, D)), _full_spec((1, 1))]
    return pl.pallas_call(
        _edge2_body,
        grid=(E_S // BE,),
        in_specs=specs,
        out_specs=[pl.BlockSpec((BR, 128), lambda i: (i, 0)),
                   pl.BlockSpec((BR, 8), lambda i: (i, 0))],
        out_shape=[jax.ShapeDtypeStruct((E_S // 8, 128), jnp.float32),
                   jax.ShapeDtypeStruct((E_S // 8, 8), jnp.float32)],
    )(xcs_pk, xcd_pk, ef_pk,
      pe["Ws"][0].reshape(D, 1), pe["bs"][0].reshape(D, 1),
      pe["gamma"].reshape(D, 1), pe["beta"].reshape(D, 1),
      w0[0:32].T, w0[32:64].T, w0[64:80].T,
      pm["bs"][0].reshape(h, 1),
      pm["Ws"][1].T, pm["bs"][1].reshape(h, 1),
      pm["Ws"][2].T, pm["bs"][2].reshape(h, 1),
      pm["Ws"][3].T, pm["bs"][3].reshape(D, 1),
      pdec["Ws"][0].T, pdec["bs"][0].reshape(D, 1),
      pdec["Ws"][1].T, pdec["bs"][1].reshape(1, 1))


def _node1_body(x, p0, p1, gf, wg, bg, gg, btg, w0, b0, w1, b1, w2, b2, out):
    agg = p0[:] + p1[:]
    g = _ln(gf[:] * wg[:] + bg[:], gg[:], btg[:])
    h = _dot(x[:], w0[0:16])
    h += _dot(agg, w0[16:32])
    h += _dot(g, w0[32:48]) + b0[:]
    h = jax.nn.relu(h)
    h = jax.nn.relu(_dot(h, w1[:]) + b1[:])
    x1 = _dot(h, w2[:]) + b2[:]
    out[:] = jnp.concatenate([x1, x[:]], axis=-1)


def _node1(x, parts, gfeat, pg, pm):
    h = pm["Ws"][1].shape[0]
    specs = [pl.BlockSpec((BN, D), lambda i: (i, 0)),
             pl.BlockSpec((BN, D), lambda i: (i, 0)),
             pl.BlockSpec((BN, D), lambda i: (i, 0)),
             _full_spec((1, 1)),
             _full_spec((1, D)), _full_spec((1, D)), _full_spec((1, D)), _full_spec((1, D)),
             _full_spec((3 * D, h)), _full_spec((1, h)),
             _full_spec((h, h)), _full_spec((1, h)),
             _full_spec((h, D)), _full_spec((1, D))]
    return pl.pallas_call(
        _node1_body,
        grid=(N // BN,),
        in_specs=specs,
        out_specs=pl.BlockSpec((BN, 2 * D), lambda i: (i, 0)),
        out_shape=jax.ShapeDtypeStruct((N, 2 * D), jnp.float32),
    )(x, parts[0], parts[1], gfeat,
      pg["Ws"][0].reshape(1, D), pg["bs"][0].reshape(1, D),
      pg["gamma"].reshape(1, D), pg["beta"].reshape(1, D),
      pm["Ws"][0], pm["bs"][0].reshape(1, h),
      pm["Ws"][1], pm["bs"][1].reshape(1, h),
      pm["Ws"][2], pm["bs"][2].reshape(1, D))


def _node2_body(xc, p0, p1, w0, b0, w1, b1, w2, b2, wn0, bn0, wn1, bn1, out):
    agg = p0[:] + p1[:]
    h = _dot(xc[:], w0[0:32])
    h += _dot(agg, w0[32:48]) + b0[:]
    h = jax.nn.relu(h)
    h = jax.nn.relu(_dot(h, w1[:]) + b1[:])
    x2 = _dot(h, w2[:]) + b2[:]
    d = jax.nn.relu(_dot(x2, wn0[:]) + bn0[:])
    out[:] = _dot(d, wn1[:]) + bn1[:]


def _node2(xc, parts, pm, pdec):
    h = pm["Ws"][1].shape[0]
    specs = [pl.BlockSpec((BN, 2 * D), lambda i: (i, 0)),
             pl.BlockSpec((BN, D), lambda i: (i, 0)),
             pl.BlockSpec((BN, D), lambda i: (i, 0)),
             _full_spec((3 * D, h)), _full_spec((1, h)),
             _full_spec((h, h)), _full_spec((1, h)),
             _full_spec((h, D)), _full_spec((1, D)),
             _full_spec((D, D)), _full_spec((1, D)),
             _full_spec((D, D_IN)), _full_spec((1, D_IN))]
    return pl.pallas_call(
        _node2_body,
        grid=(N // BN,),
        in_specs=specs,
        out_specs=pl.BlockSpec((BN, D_IN), lambda i: (i, 0)),
        out_shape=jax.ShapeDtypeStruct((N, D_IN), jnp.float32),
    )(xc, parts[0], parts[1],
      pm["Ws"][0], pm["bs"][0].reshape(1, h),
      pm["Ws"][1], pm["bs"][1].reshape(1, h),
      pm["Ws"][2], pm["bs"][2].reshape(1, D),
      pdec["Ws"][0], pdec["bs"][0].reshape(1, D),
      pdec["Ws"][1], pdec["bs"][1].reshape(1, D_IN))


@functools.lru_cache(maxsize=None)
def _make_gather(d):
    mesh = plsc.VectorSubcoreMesh(core_axis_name="c", subcore_axis_name="s",
                                  num_cores=2, num_subcores=16)

    @functools.partial(
        pl.kernel,
        out_type=[jax.ShapeDtypeStruct((E_S, d), jnp.float32),
                  jax.ShapeDtypeStruct((E_S, d), jnp.float32)],
        mesh=mesh,
        compiler_params=pltpu.CompilerParams(use_tc_tiling_on_sc=False),
        scratch_types=[pltpu.VMEM((NB, 128), jnp.int32),
                       pltpu.VMEM((CHUNK, d), jnp.float32),
                       pltpu.SemaphoreType.DMA],
    )
    def gath(tab, src_i, dst_i, out_s, out_d, idx_v, rows_v, sem):
        wid = lax.axis_index("c") * 16 + lax.axis_index("s")
        bb = wid * NBATCH
        be = wid * EPW

        def chunk(jj, idx_hbm, out_hbm):
            pltpu.sync_copy(idx_hbm.at[pl.ds(bb + jj * NB, NB)], idx_v)
            descs = [pltpu.async_copy(tab.at[idx_v.at[b]],
                                      rows_v.at[pl.ds(b * 128, 128)], sem)
                     for b in range(NB)]
            for dd in descs:
                dd.wait()
            pltpu.sync_copy(rows_v, out_hbm.at[pl.ds(be + jj * CHUNK, CHUNK)])

        def body(jj, _):
            chunk(jj, src_i, out_s)
            chunk(jj, dst_i, out_d)
            return 0

        lax.fori_loop(0, NCHUNK, body, 0)

    return gath


@functools.lru_cache(maxsize=None)
def _make_scatter():
    mesh = plsc.VectorSubcoreMesh(core_axis_name="c", subcore_axis_name="s",
                                  num_cores=2, num_subcores=16)

    @functools.partial(
        pl.kernel,
        out_type=jax.ShapeDtypeStruct((2, NROW, D), jnp.float32),
        mesh=mesh,
        compiler_params=pltpu.CompilerParams(use_tc_tiling_on_sc=False),
        scratch_types=[pltpu.VMEM((NB, 128), jnp.int32),
                       pltpu.VMEM((CHUNK, D), jnp.float32),
                       pltpu.VMEM((RPT, D), jnp.float32),
                       pltpu.VMEM_SHARED((NROW, D), jnp.float32)],
    )
    def scat(eu0, eu1, eu2, eu3, eu4, dst_i, zrow, out,
             idx_v, rows_v, zbuf, acc):
        c = lax.axis_index("c")
        s = lax.axis_index("s")
        wid = c * 16 + s

        pltpu.sync_copy(zrow.at[pl.ds(0, RPT)], zbuf)
        pltpu.sync_copy(zbuf, acc.at[pl.ds(s * RPT, RPT)])
        plsc.subcore_barrier()

        for si, eu in enumerate((eu0, eu1, eu2, eu3, eu4)):
            bb = si * IDXR + wid * NBATCH
            be = wid * EPW

            def body(jj, _, eu=eu, bb=bb, be=be):
                pltpu.sync_copy(dst_i.at[pl.ds(bb + jj * NB, NB)], idx_v)
                pltpu.sync_copy(eu.at[pl.ds(be + jj * CHUNK, CHUNK)], rows_v)
                for b in range(NB):
                    pltpu.sync_copy(rows_v.at[pl.ds(b * 128, 128)],
                                    acc.at[idx_v.at[b]], add=True)
                return 0

            lax.fori_loop(0, NCHUNK, body, 0)
        plsc.subcore_barrier()

        pltpu.sync_copy(acc.at[pl.ds(s * RPT, RPT)], zbuf)
        pltpu.sync_copy(zbuf, out.at[c, pl.ds(s * RPT, RPT)])

    return scat


def kernel(node_features, pred_edge_features, assoc_edge_features,
           global_features, params, pred_edge_index, assoc_edge_index):
    f32 = jnp.float32
    pad = E_PAD - E

    src1 = jnp.concatenate([pred_edge_index[0], jnp.zeros((pad,), jnp.int32)])
    dst1 = jnp.concatenate([pred_edge_index[1],
                            jnp.full((pad,), DUMP, jnp.int32)])
    src2 = jnp.concatenate([assoc_edge_index[0], jnp.zeros((pad,), jnp.int32)])
    dst2 = jnp.concatenate([assoc_edge_index[1],
                            jnp.full((pad,), DUMP, jnp.int32)])
    src1 = src1.reshape(E_PAD // 128, 128)
    dst1 = dst1.reshape(E_PAD // 128, 128)
    src2 = src2.reshape(E_PAD // 128, 128)
    dst2 = dst2.reshape(E_PAD // 128, 128)

    pef = jnp.concatenate([pred_edge_features, jnp.zeros((pad, 1), f32)])
    aef = jnp.concatenate([assoc_edge_features, jnp.zeros((pad, 1), f32)])
    pef = pef.reshape(E_PAD // 8, 8)
    aef = aef.reshape(E_PAD // 4, 4)
    zrow = jnp.zeros((RPT, D), f32)

    nblk = E_PAD // BE
    dst2p = (dst2.reshape(-1).reshape(nblk, 2, BE // 8, 4)
             .transpose(0, 2, 3, 1).reshape(E_PAD // 128, 128))

    x = _enc_x(node_features, params["node_enc"])

    g16 = _make_gather(D)
    g32 = _make_gather(2 * D)
    scat = _make_scatter()

    e1u = []
    for si in range(NSLICE):
        r0 = si * IDXR
        xs, xd = g16(x, src1[r0:r0 + IDXR], dst1[r0:r0 + IDXR])
        e1u.append(_edge1(xs.reshape(E_S // 8, 128),
                          xd.reshape(E_S // 8, 128),
                          pef[si * (E_S // 8):(si + 1) * (E_S // 8)],
                          global_features, params["pe_enc"],
                          params["g_enc"], params["tgl1_edge"]))
    parts1 = scat(*[e.reshape(E_S, D) for e in e1u], dst1, zrow)[:, :N, :]
    xc = _node1(x, parts1, global_features, params["g_enc"],
                params["tgl1_node"])

    e2u, eo = [], []
    for si in range(NSLICE):
        r0 = si * IDXR
        xcs, xcd = g32(xc, src2[r0:r0 + IDXR], dst2[r0:r0 + IDXR])
        a, b = _edge2(xcs.reshape(E_S // 4, 128),
                      xcd.reshape(E_S // 4, 128),
                      aef[si * (E_S // 4):(si + 1) * (E_S // 4)],
                      params["ae_enc"], params["tgl2_edge"],
                      params["edge_dec"])
        e2u.append(a)
        eo.append(b)
    parts2 = scat(*[e.reshape(E_S, D) for e in e2u], dst2p, zrow)[:, :N, :]
    nodes_out = _node2(xc, parts2, params["tgl2_node"], params["node_dec"])

    eo = jnp.concatenate(eo)
    eo = (eo.reshape(nblk, BE // 8, 4, 2).transpose(0, 3, 1, 2)
          .reshape(E_PAD, 1))
    return nodes_out, eo[:E]


# speedup vs baseline: 4.5505x; 1.2677x over previous
import functools

import jax
import jax.numpy as jnp
from jax import lax
from jax.experimental import pallas as pl
from jax.experimental.pallas import tpu as pltpu
from jax.experimental.pallas import tpu_sc as plsc

N = 50000
E = 800000
D_IN = 6
D = 16

NW = 32
NSLICE = 5
E_PAD = 819200
E_S = E_PAD // NSLICE
EPW = E_S // NW
NBATCH = EPW // 128
NB = 8
NCHUNK = NBATCH // NB
CHUNK = NB * 128
IDXR = E_S // 128

NROW = 50048
DUMP = 50000
RPT = NROW // 16

BE = 8192
BR = BE // 8
BR2 = BE // 4
BN = 2000


def _full_spec(shape):
    nd = len(shape)
    return pl.BlockSpec(shape, lambda i: (0,) * nd)


def _dot(a, w):
    return jnp.dot(a, w, preferred_element_type=jnp.float32)


def _ln(y, gamma, beta):
    mu = jnp.mean(y, axis=-1, keepdims=True)
    var = jnp.mean((y - mu) ** 2, axis=-1, keepdims=True)
    return (y - mu) * lax.rsqrt(var + 1e-5) * gamma + beta


def _enc_x_body(nf, w, b, gam, bet, out):
    out[:] = _ln(_dot(nf[:], w[:]) + b[:], gam[:], bet[:])


def _enc_x(node_features, p):
    return pl.pallas_call(
        _enc_x_body,
        grid=(N // BN,),
        in_specs=[pl.BlockSpec((BN, D_IN), lambda i: (i, 0)),
                  _full_spec((D_IN, D)), _full_spec((1, D)),
                  _full_spec((1, D)), _full_spec((1, D))],
        out_specs=pl.BlockSpec((BN, D), lambda i: (i, 0)),
        out_shape=jax.ShapeDtypeStruct((N, D), jnp.float32),
    )(node_features, p["Ws"][0], p["bs"][0].reshape(1, D),
      p["gamma"].reshape(1, D), p["beta"].reshape(1, D))


def _ln_t(y, gamma, beta):
    mu = jnp.mean(y, axis=0, keepdims=True)
    var = jnp.mean((y - mu) ** 2, axis=0, keepdims=True)
    return (y - mu) * lax.rsqrt(var + 1e-5) * gamma + beta


def _edge1_body(xs, xd, ef, gf, we, be_, ge_, bt_, wg, bg, gg, btg,
                w0x, w0d, w0e, w0g, b0, w1, b1, w2, b2, w3, b3, out):
    xst = jnp.swapaxes(xs[:], 0, 1)
    xdt = jnp.swapaxes(xd[:], 0, 1)
    pt = jnp.swapaxes(ef[:], 0, 1)
    g = _ln_t(gf[:] * wg[:] + bg[:], gg[:], btg[:])
    gc = _dot(w0g[:], g)
    subs = []
    for a in range(8):
        e1 = _ln_t(pt[a:a + 1, :] * we[:] + be_[:], ge_[:], bt_[:])
        h = _dot(w0x[:], xst[16 * a:16 * a + 16, :])
        h += _dot(w0d[:], xdt[16 * a:16 * a + 16, :])
        h += _dot(w0e[:], e1)
        h += gc + b0[:]
        h = jax.nn.relu(h)
        h = jax.nn.relu(_dot(w1[:], h) + b1[:])
        h = jax.nn.relu(_dot(w2[:], h) + b2[:])
        subs.append(_dot(w3[:], h) + b3[:])
    out[:] = jnp.swapaxes(jnp.concatenate(subs, axis=0), 0, 1)


def _edge1(xs_pk, xd_pk, ef_pk, gfeat, pe, pg, pm):
    h = pm["Ws"][1].shape[0]
    w0 = pm["Ws"][0]
    specs = [pl.BlockSpec((BR, 128), lambda i: (i, 0)),
             pl.BlockSpec((BR, 128), lambda i: (i, 0)),
             pl.BlockSpec((BR, 8), lambda i: (i, 0)),
             _full_spec((1, 1)),
             _full_spec((D, 1)), _full_spec((D, 1)), _full_spec((D, 1)), _full_spec((D, 1)),
             _full_spec((D, 1)), _full_spec((D, 1)), _full_spec((D, 1)), _full_spec((D, 1)),
             _full_spec((h, D)), _full_spec((h, D)), _full_spec((h, D)), _full_spec((h, D)),
             _full_spec((h, 1)),
             _full_spec((h, h)), _full_spec((h, 1)),
             _full_spec((h, h)), _full_spec((h, 1)),
             _full_spec((D, h)), _full_spec((D, 1))]
    return pl.pallas_call(
        _edge1_body,
        grid=(E_S // BE,),
        in_specs=specs,
        out_specs=pl.BlockSpec((BR, 128), lambda i: (i, 0)),
        out_shape=jax.ShapeDtypeStruct((E_S // 8, 128), jnp.float32),
    )(xs_pk, xd_pk, ef_pk, gfeat,
      pe["Ws"][0].reshape(D, 1), pe["bs"][0].reshape(D, 1),
      pe["gamma"].reshape(D, 1), pe["beta"].reshape(D, 1),
      pg["Ws"][0].reshape(D, 1), pg["bs"][0].reshape(D, 1),
      pg["gamma"].reshape(D, 1), pg["beta"].reshape(D, 1),
      w0[0:16].T, w0[16:32].T, w0[32:48].T, w0[48:64].T,
      pm["bs"][0].reshape(h, 1),
      pm["Ws"][1].T, pm["bs"][1].reshape(h, 1),
      pm["Ws"][2].T, pm["bs"][2].reshape(h, 1),
      pm["Ws"][3].T, pm["bs"][3].reshape(D, 1))


def _edge2_body(xsa, xsb, xda, xdb, ef, we, be_, ge_, bt_,
                w0sa, w0sb, w0da, w0db, w0e, b0, w1, b1, w2, b2, w3, b3,
                wd0, bd0, wd1, bd1, out_e, out_d):
    xsat = jnp.swapaxes(xsa[:], 0, 1)
    xsbt = jnp.swapaxes(xsb[:], 0, 1)
    xdat = jnp.swapaxes(xda[:], 0, 1)
    xdbt = jnp.swapaxes(xdb[:], 0, 1)
    pt = jnp.swapaxes(ef[:], 0, 1)
    eus, eos = [], []
    for a in range(8):
        sl = slice(16 * a, 16 * a + 16)
        e2 = _ln_t(pt[a:a + 1, :] * we[:] + be_[:], ge_[:], bt_[:])
        h = _dot(w0sa[:], xsat[sl, :])
        h += _dot(w0sb[:], xsbt[sl, :])
        h += _dot(w0da[:], xdat[sl, :])
        h += _dot(w0db[:], xdbt[sl, :])
        h += _dot(w0e[:], e2) + b0[:]
        h = jax.nn.relu(h)
        h = jax.nn.relu(_dot(w1[:], h) + b1[:])
        h = jax.nn.relu(_dot(w2[:], h) + b2[:])
        eu = _dot(w3[:], h) + b3[:]
        d = jax.nn.relu(_dot(wd0[:], eu) + bd0[:])
        eos.append(_dot(wd1[:], d) + bd1[:])
        eus.append(eu)
    out_e[:] = jnp.swapaxes(jnp.concatenate(eus, axis=0), 0, 1)
    out_d[:] = jnp.swapaxes(jnp.concatenate(eos, axis=0), 0, 1)


def _edge2(xsa, xsb, xda, xdb, ef_pk, pe, pm, pdec):
    h = pm["Ws"][1].shape[0]
    w0 = pm["Ws"][0]
    specs = [pl.BlockSpec((BR, 128), lambda i: (i, 0)),
             pl.BlockSpec((BR, 128), lambda i: (i, 0)),
             pl.BlockSpec((BR, 128), lambda i: (i, 0)),
             pl.BlockSpec((BR, 128), lambda i: (i, 0)),
             pl.BlockSpec((BR, 8), lambda i: (i, 0)),
             _full_spec((D, 1)), _full_spec((D, 1)), _full_spec((D, 1)), _full_spec((D, 1)),
             _full_spec((h, D)), _full_spec((h, D)), _full_spec((h, D)),
             _full_spec((h, D)), _full_spec((h, D)),
             _full_spec((h, 1)),
             _full_spec((h, h)), _full_spec((h, 1)),
             _full_spec((h, h)), _full_spec((h, 1)),
             _full_spec((D, h)), _full_spec((D, 1)),
             _full_spec((D, D)), _full_spec((D, 1)),
             _full_spec((1, D)), _full_spec((1, 1))]
    return pl.pallas_call(
        _edge2_body,
        grid=(E_S // BE,),
        in_specs=specs,
        out_specs=[pl.BlockSpec((BR, 128), lambda i: (i, 0)),
                   pl.BlockSpec((BR, 8), lambda i: (i, 0))],
        out_shape=[jax.ShapeDtypeStruct((E_S // 8, 128), jnp.float32),
                   jax.ShapeDtypeStruct((E_S // 8, 8), jnp.float32)],
    )(xsa, xsb, xda, xdb, ef_pk,
      pe["Ws"][0].reshape(D, 1), pe["bs"][0].reshape(D, 1),
      pe["gamma"].reshape(D, 1), pe["beta"].reshape(D, 1),
      w0[0:16].T, w0[16:32].T, w0[32:48].T, w0[48:64].T, w0[64:80].T,
      pm["bs"][0].reshape(h, 1),
      pm["Ws"][1].T, pm["bs"][1].reshape(h, 1),
      pm["Ws"][2].T, pm["bs"][2].reshape(h, 1),
      pm["Ws"][3].T, pm["bs"][3].reshape(D, 1),
      pdec["Ws"][0].T, pdec["bs"][0].reshape(D, 1),
      pdec["Ws"][1].T, pdec["bs"][1].reshape(1, 1))


def _node1_body(x, p0, p1, gf, wg, bg, gg, btg, w0, b0, w1, b1, w2, b2, out):
    agg = p0[:] + p1[:]
    g = _ln(gf[:] * wg[:] + bg[:], gg[:], btg[:])
    h = _dot(x[:], w0[0:16])
    h += _dot(agg, w0[16:32])
    h += _dot(g, w0[32:48]) + b0[:]
    h = jax.nn.relu(h)
    h = jax.nn.relu(_dot(h, w1[:]) + b1[:])
    out[:] = _dot(h, w2[:]) + b2[:]


def _node1(x, parts, gfeat, pg, pm):
    h = pm["Ws"][1].shape[0]
    specs = [pl.BlockSpec((BN, D), lambda i: (i, 0)),
             pl.BlockSpec((BN, D), lambda i: (i, 0)),
             pl.BlockSpec((BN, D), lambda i: (i, 0)),
             _full_spec((1, 1)),
             _full_spec((1, D)), _full_spec((1, D)), _full_spec((1, D)), _full_spec((1, D)),
             _full_spec((3 * D, h)), _full_spec((1, h)),
             _full_spec((h, h)), _full_spec((1, h)),
             _full_spec((h, D)), _full_spec((1, D))]
    return pl.pallas_call(
        _node1_body,
        grid=(N // BN,),
        in_specs=specs,
        out_specs=pl.BlockSpec((BN, D), lambda i: (i, 0)),
        out_shape=jax.ShapeDtypeStruct((N, D), jnp.float32),
    )(x, parts[0], parts[1], gfeat,
      pg["Ws"][0].reshape(1, D), pg["bs"][0].reshape(1, D),
      pg["gamma"].reshape(1, D), pg["beta"].reshape(1, D),
      pm["Ws"][0], pm["bs"][0].reshape(1, h),
      pm["Ws"][1], pm["bs"][1].reshape(1, h),
      pm["Ws"][2], pm["bs"][2].reshape(1, D))


def _node2_body(x1, x, p0, p1, w0, b0, w1, b1, w2, b2, wn0, bn0, wn1, bn1,
                out):
    agg = p0[:] + p1[:]
    h = _dot(x1[:], w0[0:16])
    h += _dot(x[:], w0[16:32])
    h += _dot(agg, w0[32:48]) + b0[:]
    h = jax.nn.relu(h)
    h = jax.nn.relu(_dot(h, w1[:]) + b1[:])
    x2 = _dot(h, w2[:]) + b2[:]
    d = jax.nn.relu(_dot(x2, wn0[:]) + bn0[:])
    out[:] = _dot(d, wn1[:]) + bn1[:]


def _node2(x1, x, parts, pm, pdec):
    h = pm["Ws"][1].shape[0]
    specs = [pl.BlockSpec((BN, D), lambda i: (i, 0)),
             pl.BlockSpec((BN, D), lambda i: (i, 0)),
             pl.BlockSpec((BN, D), lambda i: (i, 0)),
             pl.BlockSpec((BN, D), lambda i: (i, 0)),
             _full_spec((3 * D, h)), _full_spec((1, h)),
             _full_spec((h, h)), _full_spec((1, h)),
             _full_spec((h, D)), _full_spec((1, D)),
             _full_spec((D, D)), _full_spec((1, D)),
             _full_spec((D, D_IN)), _full_spec((1, D_IN))]
    return pl.pallas_call(
        _node2_body,
        grid=(N // BN,),
        in_specs=specs,
        out_specs=pl.BlockSpec((BN, D_IN), lambda i: (i, 0)),
        out_shape=jax.ShapeDtypeStruct((N, D_IN), jnp.float32),
    )(x1, x, parts[0], parts[1],
      pm["Ws"][0], pm["bs"][0].reshape(1, h),
      pm["Ws"][1], pm["bs"][1].reshape(1, h),
      pm["Ws"][2], pm["bs"][2].reshape(1, D),
      pdec["Ws"][0], pdec["bs"][0].reshape(1, D),
      pdec["Ws"][1], pdec["bs"][1].reshape(1, D_IN))


@functools.lru_cache(maxsize=None)
def _make_gather(d):
    mesh = plsc.VectorSubcoreMesh(core_axis_name="c", subcore_axis_name="s",
                                  num_cores=2, num_subcores=16)

    @functools.partial(
        pl.kernel,
        out_type=[jax.ShapeDtypeStruct((E_S, d), jnp.float32),
                  jax.ShapeDtypeStruct((E_S, d), jnp.float32)],
        mesh=mesh,
        compiler_params=pltpu.CompilerParams(use_tc_tiling_on_sc=False),
        scratch_types=[pltpu.VMEM((NB, 128), jnp.int32),
                       pltpu.VMEM((CHUNK, d), jnp.float32),
                       pltpu.SemaphoreType.DMA],
    )
    def gath(tab, src_i, dst_i, out_s, out_d, idx_v, rows_v, sem):
        wid = lax.axis_index("c") * 16 + lax.axis_index("s")
        bb = wid * NBATCH
        be = wid * EPW

        def chunk(jj, idx_hbm, out_hbm):
            pltpu.sync_copy(idx_hbm.at[pl.ds(bb + jj * NB, NB)], idx_v)
            descs = [pltpu.async_copy(tab.at[idx_v.at[b]],
                                      rows_v.at[pl.ds(b * 128, 128)], sem)
                     for b in range(NB)]
            for dd in descs:
                dd.wait()
            pltpu.sync_copy(rows_v, out_hbm.at[pl.ds(be + jj * CHUNK, CHUNK)])

        def body(jj, _):
            chunk(jj, src_i, out_s)
            chunk(jj, dst_i, out_d)
            return 0

        lax.fori_loop(0, NCHUNK, body, 0)

    return gath


@functools.lru_cache(maxsize=None)
def _make_gather2():
    mesh = plsc.VectorSubcoreMesh(core_axis_name="c", subcore_axis_name="s",
                                  num_cores=2, num_subcores=16)

    @functools.partial(
        pl.kernel,
        out_type=[jax.ShapeDtypeStruct((E_S, D), jnp.float32)] * 4,
        mesh=mesh,
        compiler_params=pltpu.CompilerParams(use_tc_tiling_on_sc=False),
        scratch_types=[pltpu.VMEM((NB, 128), jnp.int32),
                       pltpu.VMEM((CHUNK, D), jnp.float32),
                       pltpu.SemaphoreType.DMA],
    )
    def gath2(tab1, tab2, src_i, dst_i, o_s1, o_s2, o_d1, o_d2,
              idx_v, rows_v, sem):
        wid = lax.axis_index("c") * 16 + lax.axis_index("s")
        bb = wid * NBATCH
        be = wid * EPW

        def tabcopy(jj, tab, out_hbm):
            descs = [pltpu.async_copy(tab.at[idx_v.at[b]],
                                      rows_v.at[pl.ds(b * 128, 128)], sem)
                     for b in range(NB)]
            for dd in descs:
                dd.wait()
            pltpu.sync_copy(rows_v, out_hbm.at[pl.ds(be + jj * CHUNK, CHUNK)])

        def body(jj, _):
            pltpu.sync_copy(src_i.at[pl.ds(bb + jj * NB, NB)], idx_v)
            tabcopy(jj, tab1, o_s1)
            tabcopy(jj, tab2, o_s2)
            pltpu.sync_copy(dst_i.at[pl.ds(bb + jj * NB, NB)], idx_v)
            tabcopy(jj, tab1, o_d1)
            tabcopy(jj, tab2, o_d2)
            return 0

        lax.fori_loop(0, NCHUNK, body, 0)

    return gath2


@functools.lru_cache(maxsize=None)
def _make_scatter():
    mesh = plsc.VectorSubcoreMesh(core_axis_name="c", subcore_axis_name="s",
                                  num_cores=2, num_subcores=16)

    @functools.partial(
        pl.kernel,
        out_type=jax.ShapeDtypeStruct((2, NROW, D), jnp.float32),
        mesh=mesh,
        compiler_params=pltpu.CompilerParams(use_tc_tiling_on_sc=False),
        scratch_types=[pltpu.VMEM((NB, 128), jnp.int32),
                       pltpu.VMEM((CHUNK, D), jnp.float32),
                       pltpu.VMEM((RPT, D), jnp.float32),
                       pltpu.VMEM_SHARED((NROW, D), jnp.float32)],
    )
    def scat(eu0, eu1, eu2, eu3, eu4, dst_i, zrow, out,
             idx_v, rows_v, zbuf, acc):
        c = lax.axis_index("c")
        s = lax.axis_index("s")
        wid = c * 16 + s

        pltpu.sync_copy(zrow.at[pl.ds(0, RPT)], zbuf)
        pltpu.sync_copy(zbuf, acc.at[pl.ds(s * RPT, RPT)])
        plsc.subcore_barrier()

        for si, eu in enumerate((eu0, eu1, eu2, eu3, eu4)):
            bb = si * IDXR + wid * NBATCH
            be = wid * EPW

            def body(jj, _, eu=eu, bb=bb, be=be):
                pltpu.sync_copy(dst_i.at[pl.ds(bb + jj * NB, NB)], idx_v)
                pltpu.sync_copy(eu.at[pl.ds(be + jj * CHUNK, CHUNK)], rows_v)
                for b in range(NB):
                    pltpu.sync_copy(rows_v.at[pl.ds(b * 128, 128)],
                                    acc.at[idx_v.at[b]], add=True)
                return 0

            lax.fori_loop(0, NCHUNK, body, 0)
        plsc.subcore_barrier()

        pltpu.sync_copy(acc.at[pl.ds(s * RPT, RPT)], zbuf)
        pltpu.sync_copy(zbuf, out.at[c, pl.ds(s * RPT, RPT)])

    return scat


def kernel(node_features, pred_edge_features, assoc_edge_features,
           global_features, params, pred_edge_index, assoc_edge_index):
    f32 = jnp.float32
    pad = E_PAD - E

    src1 = jnp.concatenate([pred_edge_index[0], jnp.zeros((pad,), jnp.int32)])
    dst1 = jnp.concatenate([pred_edge_index[1],
                            jnp.full((pad,), DUMP, jnp.int32)])
    src2 = jnp.concatenate([assoc_edge_index[0], jnp.zeros((pad,), jnp.int32)])
    dst2 = jnp.concatenate([assoc_edge_index[1],
                            jnp.full((pad,), DUMP, jnp.int32)])
    src1 = src1.reshape(E_PAD // 128, 128)
    dst1 = dst1.reshape(E_PAD // 128, 128)
    src2 = src2.reshape(E_PAD // 128, 128)
    dst2 = dst2.reshape(E_PAD // 128, 128)

    pef = jnp.concatenate([pred_edge_features, jnp.zeros((pad, 1), f32)])
    aef = jnp.concatenate([assoc_edge_features, jnp.zeros((pad, 1), f32)])
    pef = pef.reshape(E_PAD // 8, 8)
    aef = aef.reshape(E_PAD // 8, 8)
    zrow = jnp.zeros((RPT, D), f32)

    x = _enc_x(node_features, params["node_enc"])

    g16 = _make_gather(D)
    g2 = _make_gather2()
    scat = _make_scatter()

    e1u = []
    for si in range(NSLICE):
        r0 = si * IDXR
        xs, xd = g16(x, src1[r0:r0 + IDXR], dst1[r0:r0 + IDXR])
        e1u.append(_edge1(xs.reshape(E_S // 8, 128),
                          xd.reshape(E_S // 8, 128),
                          pef[si * (E_S // 8):(si + 1) * (E_S // 8)],
                          global_features, params["pe_enc"],
                          params["g_enc"], params["tgl1_edge"]))
    parts1 = scat(*[e.reshape(E_S, D) for e in e1u], dst1, zrow)[:, :N, :]
    x1 = _node1(x, parts1, global_features, params["g_enc"],
                params["tgl1_node"])

    e2u, eo = [], []
    for si in range(NSLICE):
        r0 = si * IDXR
        xsa, xsb, xda, xdb = g2(x1, x, src2[r0:r0 + IDXR],
                                dst2[r0:r0 + IDXR])
        a, b = _edge2(xsa.reshape(E_S // 8, 128),
                      xsb.reshape(E_S // 8, 128),
                      xda.reshape(E_S // 8, 128),
                      xdb.reshape(E_S // 8, 128),
                      aef[si * (E_S // 8):(si + 1) * (E_S // 8)],
                      params["ae_enc"], params["tgl2_edge"],
                      params["edge_dec"])
        e2u.append(a)
        eo.append(b)
    parts2 = scat(*[e.reshape(E_S, D) for e in e2u], dst2, zrow)[:, :N, :]
    nodes_out = _node2(x1, x, parts2, params["tgl2_node"],
                       params["node_dec"])

    eo = jnp.concatenate(eo).reshape(E_PAD, 1)
    return nodes_out, eo[:E]
